# baseline (speedup 1.0000x reference)
import functools

import jax
import jax.numpy as jnp
from jax import lax
from jax.experimental import pallas as pl
from jax.experimental.pallas import tpu as pltpu
from jax.experimental.pallas import tpu_sc as plsc

N_TOK = 2048
K = 1024
D = 64
J = 4
F = N_TOK * J
NW = 32
B_PER_W = F // NW
DPAD = 128
SCALE = float(1 << 20)
IMAX = 2147483647


def _topj_kernel(r_ref, emb_ref, cand_ref, pad_ref):
    r = r_ref[...]
    emb = emb_ref[...]
    dots = lax.dot_general(r, emb, (((1,), (1,)), ((), ())),
                           preferred_element_type=jnp.float32,
                           precision=jax.lax.Precision.HIGHEST)
    esq = emb * emb
    e2 = lax.dot_general(jnp.ones((1, D), jnp.float32), esq,
                         (((1,), (1,)), ((), ())),
                         preferred_element_type=jnp.float32)
    dist = e2 - 2.0 * dots
    iota = jax.lax.broadcasted_iota(jnp.int32, dist.shape, 1)
    packed = (dist * SCALE).astype(jnp.int32) * K + iota
    for j in range(J):
        m = jnp.min(packed, axis=1, keepdims=True)
        cand_ref[:, j:j + 1] = m & (K - 1)
        packed = jnp.where(packed == m, IMAX, packed)
    pad_ref[:, 0:D] = emb
    pad_ref[:, D:DPAD] = jnp.zeros((K, DPAD - D), jnp.float32)


def _topj(residual, emb):
    return pl.pallas_call(
        _topj_kernel,
        out_shape=[
            jax.ShapeDtypeStruct((N_TOK, J), jnp.int32),
            jax.ShapeDtypeStruct((K, DPAD), jnp.float32),
        ],
    )(residual, emb)


def _sc_gather_kernel(emb_hbm, idx_hbm, out_hbm, idx_v, rows_v, sem):
    wid = lax.axis_index("s") * 2 + lax.axis_index("c")
    base = wid * B_PER_W
    pltpu.sync_copy(idx_hbm.at[pl.ds(base, B_PER_W)], idx_v)
    pltpu.async_copy(emb_hbm.at[idx_v], rows_v, sem).wait()
    pltpu.sync_copy(rows_v, out_hbm.at[pl.ds(base, B_PER_W)])


def _sc_gather(emb_padded, cand_flat):
    mesh = plsc.VectorSubcoreMesh(core_axis_name="c", subcore_axis_name="s")
    fn = functools.partial(
        pl.kernel,
        mesh=mesh,
        out_type=jax.ShapeDtypeStruct((F, DPAD), jnp.float32),
        scratch_types=[
            pltpu.VMEM((B_PER_W,), jnp.int32),
            pltpu.VMEM((B_PER_W, DPAD), jnp.float32),
            pltpu.SemaphoreType.DMA,
        ],
    )(_sc_gather_kernel)
    return fn(emb_padded, cand_flat)


def _rescore_kernel(r_ref, rows_ref, cand_ref, q_ref, codes_ref):
    r = r_ref[...]
    rows_pad = rows_ref[...]
    rows2 = jnp.concatenate(
        [rows_pad[:, j * DPAD:j * DPAD + D] for j in range(J)], axis=1)
    rr = jnp.concatenate([r] * J, axis=1)
    diff = rr - rows2
    sqt = (diff * diff).T
    candt = cand_ref[...].T
    best_d = None
    best_k = None
    best_j = None
    for j in range(J):
        sq = sqt[j * D:(j + 1) * D, :]
        dist_j = None
        for v in range(D // 8):
            g = sq[8 * v:8 * v + 8, :]
            a = g[0:4, :] + g[4:8, :]
            b = a[0:2, :] + a[2:4, :]
            gv = b[0:1, :] + b[1:2, :]
            dist_j = gv if dist_j is None else dist_j + gv
        k_j = candt[j:j + 1, :]
        if j == 0:
            best_d, best_k = dist_j, k_j
            best_j = jnp.zeros_like(k_j)
        else:
            take = (dist_j < best_d) | ((dist_j == best_d) & (k_j < best_k))
            best_d = jnp.where(take, dist_j, best_d)
            best_k = jnp.where(take, k_j, best_k)
            best_j = jnp.where(take, jnp.int32(j), best_j)
    codes_ref[...] = best_k
    best_j_col = best_j.T
    q = rows2[:, 0:D]
    for j in range(1, J):
        q = jnp.where(best_j_col == j, rows2[:, j * D:(j + 1) * D], q)
    q_ref[...] = q


def _rescore(residual, rows2, cand):
    return pl.pallas_call(
        _rescore_kernel,
        out_shape=[
            jax.ShapeDtypeStruct((N_TOK, D), jnp.float32),
            jax.ShapeDtypeStruct((1, N_TOK), jnp.int32),
        ],
    )(residual, rows2, cand)


def kernel(residual, emb):
    cand, emb_padded = _topj(residual, emb)
    cand_flat = cand.reshape(F)
    rows = _sc_gather(emb_padded, cand_flat)
    rows2 = rows.reshape(N_TOK, J * DPAD)
    q, codes = _rescore(residual, rows2, cand)
    return (q, codes.reshape(N_TOK))


# speedup vs baseline: 1.2728x; 1.2728x over previous
import functools

import jax
import jax.numpy as jnp
from jax import lax
from jax.experimental import pallas as pl
from jax.experimental.pallas import tpu as pltpu
from jax.experimental.pallas import tpu_sc as plsc

N_TOK = 2048
K = 1024
D = 64
J = 4
F = N_TOK * J
NW = 32
B_PER_W = F // NW
DPAD = 128
SCALE = float(1 << 20)
IMAX = 2147483647


def _topj_kernel(r_ref, emb_ref, cand_ref, pad_ref):
    r = r_ref[...]
    emb = emb_ref[...]
    dots = lax.dot_general(r, emb, (((1,), (1,)), ((), ())),
                           preferred_element_type=jnp.float32)
    esq = emb * emb
    e2 = lax.dot_general(jnp.ones((1, D), jnp.float32), esq,
                         (((1,), (1,)), ((), ())),
                         preferred_element_type=jnp.float32)
    dist = e2 - 2.0 * dots
    iota = jax.lax.broadcasted_iota(jnp.int32, dist.shape, 1)
    packed = (dist * SCALE).astype(jnp.int32) * K + iota
    for j in range(J):
        m = jnp.min(packed, axis=1, keepdims=True)
        cand_ref[8 * j:8 * j + 8, :] = jnp.reshape(m & (K - 1), (8, N_TOK // 8))
        packed = jnp.where(packed == m, IMAX, packed)
    pad_ref[:, 0:D] = emb
    pad_ref[:, D:DPAD] = jnp.zeros((K, DPAD - D), jnp.float32)


def _topj(residual, emb):
    return pl.pallas_call(
        _topj_kernel,
        out_shape=[
            jax.ShapeDtypeStruct((NW, B_PER_W), jnp.int32),
            jax.ShapeDtypeStruct((K, DPAD), jnp.float32),
        ],
    )(residual, emb)


def _sc_gather_kernel(emb_hbm, idx_hbm, out_hbm, idx_v, rows_v, sem):
    wid = lax.axis_index("s") * 2 + lax.axis_index("c")
    pltpu.sync_copy(idx_hbm.at[wid], idx_v)
    pltpu.async_copy(emb_hbm.at[idx_v], rows_v, sem).wait()
    pltpu.sync_copy(rows_v, out_hbm.at[pl.ds(wid * B_PER_W, B_PER_W)])


def _sc_gather(emb_padded, cand):
    mesh = plsc.VectorSubcoreMesh(core_axis_name="c", subcore_axis_name="s")
    fn = functools.partial(
        pl.kernel,
        mesh=mesh,
        out_type=jax.ShapeDtypeStruct((F, DPAD), jnp.float32),
        scratch_types=[
            pltpu.VMEM((B_PER_W,), jnp.int32),
            pltpu.VMEM((B_PER_W, DPAD), jnp.float32),
            pltpu.SemaphoreType.DMA,
        ],
    )(_sc_gather_kernel)
    return fn(emb_padded, cand)


def _rescore_kernel(r_ref, rows_ref, cand_ref, q_ref, codes_ref):
    r = r_ref[...]
    slabs = [rows_ref[j * N_TOK:(j + 1) * N_TOK, 0:D] for j in range(J)]
    rows2 = jnp.concatenate(slabs, axis=1)
    rr = jnp.concatenate([r] * J, axis=1)
    diff = rr - rows2
    sqt = (diff * diff).T
    best_d = None
    best_k = None
    best_j = None
    for j in range(J):
        sq = sqt[j * D:(j + 1) * D, :]
        dist_j = None
        for v in range(D // 8):
            g = sq[8 * v:8 * v + 8, :]
            a = g[0:4, :] + g[4:8, :]
            b = a[0:2, :] + a[2:4, :]
            gv = b[0:1, :] + b[1:2, :]
            dist_j = gv if dist_j is None else dist_j + gv
        k_j = jnp.reshape(cand_ref[8 * j:8 * j + 8, :], (1, N_TOK))
        if j == 0:
            best_d, best_k = dist_j, k_j
            best_j = jnp.zeros_like(k_j)
        else:
            take = (dist_j < best_d) | ((dist_j == best_d) & (k_j < best_k))
            best_d = jnp.where(take, dist_j, best_d)
            best_k = jnp.where(take, k_j, best_k)
            best_j = jnp.where(take, jnp.int32(j), best_j)
    codes_ref[...] = best_k
    best_j_col = best_j.T
    q = rows2[:, 0:D]
    for j in range(1, J):
        q = jnp.where(best_j_col == j, rows2[:, j * D:(j + 1) * D], q)
    q_ref[...] = q


def _rescore(residual, rows, cand):
    return pl.pallas_call(
        _rescore_kernel,
        out_shape=[
            jax.ShapeDtypeStruct((N_TOK, D), jnp.float32),
            jax.ShapeDtypeStruct((1, N_TOK), jnp.int32),
        ],
    )(residual, rows, cand)


def kernel(residual, emb):
    cand, emb_padded = _topj(residual, emb)
    rows = _sc_gather(emb_padded, cand)
    q, codes = _rescore(residual, rows, cand)
    return (q, codes.reshape(N_TOK))


# speedup vs baseline: 1.2816x; 1.0069x over previous
import functools

import jax
import jax.numpy as jnp
from jax import lax
from jax.experimental import pallas as pl
from jax.experimental.pallas import tpu as pltpu
from jax.experimental.pallas import tpu_sc as plsc

N_TOK = 2048
K = 1024
D = 64
J = 4
F = N_TOK * J
NW = 32
B_PER_W = F // NW
DPAD = 128
SCALE = float(1 << 20)
IMAX = 2147483647


def _topj_kernel(r_ref, emb_ref, cand_ref, pad_ref):
    r = r_ref[...]
    emb = emb_ref[...]
    dots = lax.dot_general(r, emb, (((1,), (1,)), ((), ())),
                           preferred_element_type=jnp.float32)
    esq = emb * emb
    e2 = lax.dot_general(jnp.ones((1, D), jnp.float32), esq,
                         (((1,), (1,)), ((), ())),
                         preferred_element_type=jnp.float32)
    dist = e2 - 2.0 * dots
    iota = jax.lax.broadcasted_iota(jnp.int32, dist.shape, 1)
    packed = (dist * SCALE).astype(jnp.int32) * K + iota
    m = None
    for j in range(J):
        src = packed if j == 0 else jnp.where(packed <= m, IMAX, packed)
        m = jnp.min(src, axis=1, keepdims=True)
        cand_ref[8 * j:8 * j + 8, :] = jnp.reshape(m & (K - 1), (8, N_TOK // 8))
    pad_ref[:, 0:D] = emb


def _topj(residual, emb):
    return pl.pallas_call(
        _topj_kernel,
        out_shape=[
            jax.ShapeDtypeStruct((NW, B_PER_W), jnp.int32),
            jax.ShapeDtypeStruct((K, DPAD), jnp.float32),
        ],
    )(residual, emb)


def _sc_gather_kernel(emb_hbm, idx_hbm, out_hbm, idx_v, rows_v, sem):
    wid = lax.axis_index("s") * 2 + lax.axis_index("c")
    pltpu.sync_copy(idx_hbm.at[wid], idx_v)
    pltpu.async_copy(emb_hbm.at[idx_v], rows_v, sem).wait()
    pltpu.sync_copy(rows_v, out_hbm.at[pl.ds(wid * B_PER_W, B_PER_W)])


def _sc_gather(emb_padded, cand):
    mesh = plsc.VectorSubcoreMesh(core_axis_name="c", subcore_axis_name="s")
    fn = functools.partial(
        pl.kernel,
        mesh=mesh,
        out_type=jax.ShapeDtypeStruct((F, DPAD), jnp.float32),
        scratch_types=[
            pltpu.VMEM((B_PER_W,), jnp.int32),
            pltpu.VMEM((B_PER_W, DPAD), jnp.float32),
            pltpu.SemaphoreType.DMA,
        ],
    )(_sc_gather_kernel)
    return fn(emb_padded, cand)


def _rescore_kernel(r_ref, rows_ref, cand_ref, q_ref, codes_ref):
    r = r_ref[...]
    slabs = [rows_ref[j * N_TOK:(j + 1) * N_TOK, 0:D] for j in range(J)]
    rows2 = jnp.concatenate(slabs, axis=1)
    rr = jnp.concatenate([r] * J, axis=1)
    diff = rr - rows2
    sqt = (diff * diff).T
    best_d = None
    best_k = None
    best_j = None
    for j in range(J):
        sq = sqt[j * D:(j + 1) * D, :]
        dist_j = None
        for v in range(D // 8):
            g = sq[8 * v:8 * v + 8, :]
            a = g[0:4, :] + g[4:8, :]
            b = a[0:2, :] + a[2:4, :]
            gv = b[0:1, :] + b[1:2, :]
            dist_j = gv if dist_j is None else dist_j + gv
        k_j = jnp.reshape(cand_ref[8 * j:8 * j + 8, :], (1, N_TOK))
        if j == 0:
            best_d, best_k = dist_j, k_j
            best_j = jnp.zeros_like(k_j)
        else:
            take = (dist_j < best_d) | ((dist_j == best_d) & (k_j < best_k))
            best_d = jnp.where(take, dist_j, best_d)
            best_k = jnp.where(take, k_j, best_k)
            best_j = jnp.where(take, jnp.int32(j), best_j)
    codes_ref[...] = best_k
    best_j_col = best_j.T
    q = rows2[:, 0:D]
    for j in range(1, J):
        q = jnp.where(best_j_col == j, rows2[:, j * D:(j + 1) * D], q)
    q_ref[...] = q


def _rescore(residual, rows, cand):
    return pl.pallas_call(
        _rescore_kernel,
        out_shape=[
            jax.ShapeDtypeStruct((N_TOK, D), jnp.float32),
            jax.ShapeDtypeStruct((1, N_TOK), jnp.int32),
        ],
    )(residual, rows, cand)


def kernel(residual, emb):
    cand, emb_padded = _topj(residual, emb)
    rows = _sc_gather(emb_padded, cand)
    q, codes = _rescore(residual, rows, cand)
    return (q, codes.reshape(N_TOK))
